# Initial kernel scaffold; baseline (speedup 1.0000x reference)
import jax
import jax.numpy as jnp
from jax.experimental import pallas as pl


def kernel(embeddings, context_embeddings, target, context, negative_samples):
    raise NotImplementedError("write your pallas kernel here")


# speedup vs baseline: 3.7363x; 3.7363x over previous
import functools

import jax
import jax.numpy as jnp
from jax import lax
from jax.experimental import pallas as pl
from jax.experimental.pallas import tpu as pltpu
from jax.experimental.pallas import tpu_sc as plsc

DIM = 64
B = 16384
K = 20
NC, NS = 2, 16
NW = NC * NS
S = B // NW
G = 16
NG = S // G
NPG = G * K
NQ = 4
QROWS = NPG // NQ
ROWP = 17


def _sc_logits(emb, cemb, tgt, ctx, negf):
  mesh = plsc.VectorSubcoreMesh(core_axis_name="c", subcore_axis_name="s")

  @functools.partial(
      pl.kernel,
      out_type=(jax.ShapeDtypeStruct((NW, S), jnp.float32),
                jax.ShapeDtypeStruct((NW, S * K), jnp.float32)),
      mesh=mesh,
      compiler_params=pltpu.CompilerParams(
          needs_layout_passes=False, use_tc_tiling_on_sc=False),
      scratch_types=[
          pltpu.VMEM((S,), jnp.int32),
          pltpu.VMEM((S,), jnp.int32),
          pltpu.VMEM((S * K,), jnp.int32),
          pltpu.VMEM((G, DIM), jnp.bfloat16),
          pltpu.VMEM((G, DIM), jnp.bfloat16),
          pltpu.VMEM((G, DIM), jnp.bfloat16),
          pltpu.VMEM((G, DIM), jnp.bfloat16),
          pltpu.VMEM((NPG, DIM), jnp.bfloat16),
          pltpu.VMEM((NPG, DIM), jnp.bfloat16),
          pltpu.VMEM((16 * ROWP,), jnp.float32),
          pltpu.VMEM((S,), jnp.float32),
          pltpu.VMEM((S * K,), jnp.float32),
          pltpu.SemaphoreType.DMA,
          pltpu.SemaphoreType.DMA,
          pltpu.SemaphoreType.DMA,
          pltpu.SemaphoreType.DMA,
          pltpu.SemaphoreType.DMA,
          pltpu.SemaphoreType.DMA,
      ],
  )
  def k(emb_h, cemb_h, tgt_h, ctx_h, neg_h, pos_o, neg_o,
        idx_t, idx_c, idx_n, t0, t1, c0, c1, n0, n1, scr, pos_st, neg_st,
        st0, st1, sc0, sc1, sn0, sn1):
    wid = lax.axis_index("s") * NC + lax.axis_index("c")
    base = wid * S
    pltpu.sync_copy(tgt_h.at[pl.ds(base, S)], idx_t)
    pltpu.sync_copy(ctx_h.at[pl.ds(base, S)], idx_c)
    pltpu.sync_copy(neg_h.at[pl.ds(base * K, S * K)], idx_n)

    tb, cb, nb = (t0, t1), (c0, c1), (n0, n1)
    stm, scm, snm = (st0, st1), (sc0, sc1), (sn0, sn1)

    def issue(g, u):
      pltpu.async_copy(emb_h.at[idx_t.at[pl.ds(g * G, G)]], tb[u], stm[u])
      pltpu.async_copy(cemb_h.at[idx_c.at[pl.ds(g * G, G)]], cb[u], scm[u])
      for q in range(NQ):
        pltpu.async_copy(
            cemb_h.at[idx_n.at[pl.ds(g * NPG + q * QROWS, QROWS)]],
            nb[u].at[pl.ds(q * QROWS, QROWS)], snm[u])

    def wait(u):
      pltpu.make_async_copy(
          emb_h.at[idx_t.at[pl.ds(0, G)]], tb[u], stm[u]).wait()
      pltpu.make_async_copy(
          cemb_h.at[idx_c.at[pl.ds(0, G)]], cb[u], scm[u]).wait()
      for q in range(NQ):
        pltpu.make_async_copy(
            cemb_h.at[idx_n.at[pl.ds(0, QROWS)]],
            nb[u].at[pl.ds(q * QROWS, QROWS)], snm[u]).wait()

    iota = lax.iota(jnp.int32, 16)
    iotap = iota * ROWP

    def colsum():
      r = plsc.load_gather(scr, [iotap])
      for cc in range(1, 16):
        r = r + plsc.load_gather(scr, [iotap + cc])
      return r

    def row_chunks(ref, row):
      out = []
      for h in range(2):
        ab = ref[row, pl.ds(h * 32, 32)]
        out.extend(plsc.unpack(ab, format=plsc.PackFormat.INTERLEAVED))
      return out

    def dot4(a, b):
      return (a[0] * b[0] + a[1] * b[1]) + (a[2] * b[2] + a[3] * b[3])

    def compute(g, u):
      t_ref, c_ref, n_ref = tb[u], cb[u], nb[u]
      for i in range(G):
        acc = dot4(row_chunks(t_ref, i), row_chunks(c_ref, i))
        plsc.store_scatter(scr, [iota + i * ROWP], acc)
      plsc.store_scatter(pos_st, [g * G + iota], colsum())

      for sub in range(G // 4):
        b0 = sub * 4
        tch = [row_chunks(t_ref, b0 + i) for i in range(4)]
        for kc in range(K // 4):
          for cc in range(4):
            for i in range(4):
              nrow = (b0 + i) * K + kc * 4 + cc
              acc = dot4(tch[i], row_chunks(n_ref, nrow))
              plsc.store_scatter(scr, [iota + (cc * 4 + i) * ROWP], acc)
          dst = ((kc * 4 + lax.shift_right_logical(iota, 2)) * S
                 + g * G + b0 + (iota & 3))
          plsc.store_scatter(neg_st, [dst], colsum())

    issue(0, 0)
    issue(1, 1)

    @pl.loop(0, NG // 2)
    def _t(t):
      g = t * 2
      wait(0)
      compute(g, 0)

      @pl.when(g + 2 < NG)
      def _():
        issue(g + 2, 0)

      wait(1)
      compute(g + 1, 1)

      @pl.when(g + 3 < NG)
      def _():
        issue(g + 3, 1)

    pltpu.sync_copy(pos_st, pos_o.at[wid])
    pltpu.sync_copy(neg_st, neg_o.at[wid])

  return k(emb, cemb, tgt, ctx, negf)


def _loss_tc(pos2d, neg2d):
  def body(p_ref, n_ref, o_ref):
    p = p_ref[...]
    n = n_ref[...]
    lp = jnp.minimum(p, 0.0) - jnp.log1p(jnp.exp(-jnp.abs(p)))
    ln = jnp.minimum(-n, 0.0) - jnp.log1p(jnp.exp(-jnp.abs(n)))
    o_ref[0, 0] = -(jnp.sum(lp) + jnp.sum(ln)) / B

  return pl.pallas_call(
      body,
      out_shape=jax.ShapeDtypeStruct((1, 1), jnp.float32),
      out_specs=pl.BlockSpec(memory_space=pltpu.SMEM),
  )(pos2d, neg2d)


def kernel(embeddings, context_embeddings, target, context, negative_samples):
  tgt = target.astype(jnp.int32)
  ctx = context.astype(jnp.int32)
  negf = negative_samples.astype(jnp.int32).reshape(-1)
  emb_bf = embeddings.astype(jnp.bfloat16)
  cemb_bf = context_embeddings.astype(jnp.bfloat16)
  pos_l, neg_l = _sc_logits(emb_bf, cemb_bf, tgt, ctx, negf)
  loss = _loss_tc(pos_l.reshape(B // 128, 128), neg_l.reshape(B * K // 128, 128))
  return loss.reshape(())


# speedup vs baseline: 4.6489x; 1.2443x over previous
import functools

import jax
import jax.numpy as jnp
from jax import lax
from jax.experimental import pallas as pl
from jax.experimental.pallas import tpu as pltpu
from jax.experimental.pallas import tpu_sc as plsc

DIM = 64
B = 16384
K = 20
NC, NS = 2, 16
NW = NC * NS
S = B // NW
G = 16
NG = S // G
NPG = G * K
NQ = 4
QROWS = NPG // NQ
ROWP = 17


def _sc_logits(emb, cemb, tgt, ctx, negf):
  mesh = plsc.VectorSubcoreMesh(core_axis_name="c", subcore_axis_name="s")

  @functools.partial(
      pl.kernel,
      out_type=(jax.ShapeDtypeStruct((NW, S), jnp.float32),
                jax.ShapeDtypeStruct((NW, S * K), jnp.float32)),
      mesh=mesh,
      compiler_params=pltpu.CompilerParams(
          needs_layout_passes=False, use_tc_tiling_on_sc=False),
      scratch_types=[
          pltpu.VMEM((S,), jnp.int32),
          pltpu.VMEM((S,), jnp.int32),
          pltpu.VMEM((S * K,), jnp.int32),
          pltpu.VMEM((G, DIM), jnp.float32),
          pltpu.VMEM((G, DIM), jnp.float32),
          pltpu.VMEM((G, DIM), jnp.float32),
          pltpu.VMEM((G, DIM), jnp.float32),
          pltpu.VMEM((NPG, DIM), jnp.float32),
          pltpu.VMEM((NPG, DIM), jnp.float32),
          pltpu.VMEM((16 * ROWP,), jnp.float32),
          pltpu.VMEM((16 * ROWP,), jnp.float32),
          pltpu.VMEM((S,), jnp.float32),
          pltpu.VMEM((S * K,), jnp.float32),
          pltpu.SemaphoreType.DMA,
          pltpu.SemaphoreType.DMA,
          pltpu.SemaphoreType.DMA,
          pltpu.SemaphoreType.DMA,
          pltpu.SemaphoreType.DMA,
          pltpu.SemaphoreType.DMA,
      ],
  )
  def k(emb_h, cemb_h, tgt_h, ctx_h, neg_h, pos_o, neg_o,
        idx_t, idx_c, idx_n, t0, t1, c0, c1, n0, n1, scra, scrb, pos_st, neg_st,
        st0, st1, sc0, sc1, sn0, sn1):
    wid = lax.axis_index("s") * NC + lax.axis_index("c")
    base = wid * S
    pltpu.sync_copy(tgt_h.at[pl.ds(base, S)], idx_t)
    pltpu.sync_copy(ctx_h.at[pl.ds(base, S)], idx_c)
    pltpu.sync_copy(neg_h.at[pl.ds(base * K, S * K)], idx_n)

    tb, cb, nb = (t0, t1), (c0, c1), (n0, n1)
    stm, scm, snm = (st0, st1), (sc0, sc1), (sn0, sn1)

    def issue(g, u):
      pltpu.async_copy(emb_h.at[idx_t.at[pl.ds(g * G, G)]], tb[u], stm[u])
      pltpu.async_copy(cemb_h.at[idx_c.at[pl.ds(g * G, G)]], cb[u], scm[u])
      for q in range(NQ):
        pltpu.async_copy(
            cemb_h.at[idx_n.at[pl.ds(g * NPG + q * QROWS, QROWS)]],
            nb[u].at[pl.ds(q * QROWS, QROWS)], snm[u])

    def wait(u):
      pltpu.make_async_copy(
          emb_h.at[idx_t.at[pl.ds(0, G)]], tb[u], stm[u]).wait()
      pltpu.make_async_copy(
          cemb_h.at[idx_c.at[pl.ds(0, G)]], cb[u], scm[u]).wait()
      for q in range(NQ):
        pltpu.make_async_copy(
            cemb_h.at[idx_n.at[pl.ds(0, QROWS)]],
            nb[u].at[pl.ds(q * QROWS, QROWS)], snm[u]).wait()

    iota = lax.iota(jnp.int32, 16)
    iotap = iota * ROWP

    def colsum(scr):
      cols = [plsc.load_gather(scr, [iotap + cc]) for cc in range(16)]
      while len(cols) > 1:
        cols = [cols[j] + cols[j + 1] for j in range(0, len(cols), 2)]
      return cols[0]

    def row_chunks(ref, row):
      return [ref[row, pl.ds(c * 16, 16)] for c in range(4)]

    def dot4(a, b):
      return (a[0] * b[0] + a[1] * b[1]) + (a[2] * b[2] + a[3] * b[3])

    def compute(g, u):
      t_ref, c_ref, n_ref = tb[u], cb[u], nb[u]
      for i in range(G):
        acc = dot4(row_chunks(t_ref, i), row_chunks(c_ref, i))
        plsc.store_scatter(scra, [iota + i * ROWP], acc)
      plsc.store_scatter(pos_st, [g * G + iota], colsum(scra))

      for sub in range(G // 4):
        b0 = sub * 4
        tch = [row_chunks(t_ref, b0 + i) for i in range(4)]
        for kc in range(K // 4):
          scr = scrb if kc % 2 else scra
          for cc in range(4):
            for i in range(4):
              nrow = (b0 + i) * K + kc * 4 + cc
              acc = dot4(tch[i], row_chunks(n_ref, nrow))
              plsc.store_scatter(scr, [iota + (cc * 4 + i) * ROWP], acc)
          dst = ((kc * 4 + lax.shift_right_logical(iota, 2)) * S
                 + g * G + b0 + (iota & 3))
          plsc.store_scatter(neg_st, [dst], colsum(scr))

    issue(0, 0)
    issue(1, 1)

    @pl.loop(0, NG // 2)
    def _t(t):
      g = t * 2
      wait(0)
      compute(g, 0)

      @pl.when(g + 2 < NG)
      def _():
        issue(g + 2, 0)

      wait(1)
      compute(g + 1, 1)

      @pl.when(g + 3 < NG)
      def _():
        issue(g + 3, 1)

    pltpu.sync_copy(pos_st, pos_o.at[wid])
    pltpu.sync_copy(neg_st, neg_o.at[wid])

  return k(emb, cemb, tgt, ctx, negf)


def _loss_tc(pos2d, neg2d):
  def body(p_ref, n_ref, o_ref):
    p = p_ref[...]
    n = n_ref[...]
    lp = jnp.minimum(p, 0.0) - jnp.log1p(jnp.exp(-jnp.abs(p)))
    ln = jnp.minimum(-n, 0.0) - jnp.log1p(jnp.exp(-jnp.abs(n)))
    o_ref[0, 0] = -(jnp.sum(lp) + jnp.sum(ln)) / B

  return pl.pallas_call(
      body,
      out_shape=jax.ShapeDtypeStruct((1, 1), jnp.float32),
      out_specs=pl.BlockSpec(memory_space=pltpu.SMEM),
  )(pos2d, neg2d)


def kernel(embeddings, context_embeddings, target, context, negative_samples):
  tgt = target.astype(jnp.int32)
  ctx = context.astype(jnp.int32)
  negf = negative_samples.astype(jnp.int32).reshape(-1)
  pos_l, neg_l = _sc_logits(embeddings, context_embeddings, tgt, ctx, negf)
  loss = _loss_tc(pos_l.reshape(B // 128, 128), neg_l.reshape(B * K // 128, 128))
  return loss.reshape(())


# speedup vs baseline: 5.1054x; 1.0982x over previous
import functools

import jax
import jax.numpy as jnp
from jax import lax
from jax.experimental import pallas as pl
from jax.experimental.pallas import tpu as pltpu
from jax.experimental.pallas import tpu_sc as plsc

DIM = 64
B = 16384
K = 20
NC, NS = 2, 16
NW = NC * NS
S = B // NW
G = 16
NG = S // G
NPG = G * K
NQ = 4
QROWS = NPG // NQ
ROWP = 17


def _sc_logits(emb, cemb, tgt, ctx, negf):
  mesh = plsc.VectorSubcoreMesh(core_axis_name="c", subcore_axis_name="s")

  @functools.partial(
      pl.kernel,
      out_type=(jax.ShapeDtypeStruct((NW, S), jnp.float32),
                jax.ShapeDtypeStruct((NW, S * K), jnp.float32)),
      mesh=mesh,
      compiler_params=pltpu.CompilerParams(
          needs_layout_passes=False, use_tc_tiling_on_sc=False),
      scratch_types=[
          pltpu.VMEM((S,), jnp.int32),
          pltpu.VMEM((S,), jnp.int32),
          pltpu.VMEM((S * K,), jnp.int32),
          pltpu.VMEM((G, DIM), jnp.float32),
          pltpu.VMEM((G, DIM), jnp.float32),
          pltpu.VMEM((G, DIM), jnp.float32),
          pltpu.VMEM((G, DIM), jnp.float32),
          pltpu.VMEM((NPG, DIM), jnp.float32),
          pltpu.VMEM((NPG, DIM), jnp.float32),
          pltpu.VMEM((16 * ROWP,), jnp.float32),
          pltpu.VMEM((16 * ROWP,), jnp.float32),
          pltpu.VMEM((S,), jnp.float32),
          pltpu.VMEM((S * K,), jnp.float32),
          pltpu.SemaphoreType.DMA,
          pltpu.SemaphoreType.DMA,
          pltpu.SemaphoreType.DMA,
          pltpu.SemaphoreType.DMA,
          pltpu.SemaphoreType.DMA,
          pltpu.SemaphoreType.DMA,
      ],
  )
  def k(emb_h, cemb_h, tgt_h, ctx_h, neg_h, pos_o, neg_o,
        idx_t, idx_c, idx_n, t0, t1, c0, c1, n0, n1, scra, scrb, pos_st, neg_st,
        st0, st1, sc0, sc1, sn0, sn1):
    wid = lax.axis_index("s") * NC + lax.axis_index("c")
    base = wid * S
    pltpu.sync_copy(tgt_h.at[pl.ds(base, S)], idx_t)
    pltpu.sync_copy(ctx_h.at[pl.ds(base, S)], idx_c)
    pltpu.sync_copy(neg_h.at[pl.ds(base * K, S * K)], idx_n)

    tb, cb, nb = (t0, t1), (c0, c1), (n0, n1)
    stm, scm, snm = (st0, st1), (sc0, sc1), (sn0, sn1)

    def issue(g, u):
      pltpu.async_copy(emb_h.at[idx_t.at[pl.ds(g * G, G)]], tb[u], stm[u])
      pltpu.async_copy(cemb_h.at[idx_c.at[pl.ds(g * G, G)]], cb[u], scm[u])
      for q in range(NQ):
        pltpu.async_copy(
            cemb_h.at[idx_n.at[pl.ds(g * NPG + q * QROWS, QROWS)]],
            nb[u].at[pl.ds(q * QROWS, QROWS)], snm[u])

    def wait(u):
      pltpu.make_async_copy(
          emb_h.at[idx_t.at[pl.ds(0, G)]], tb[u], stm[u]).wait()
      pltpu.make_async_copy(
          cemb_h.at[idx_c.at[pl.ds(0, G)]], cb[u], scm[u]).wait()
      for q in range(NQ):
        pltpu.make_async_copy(
            cemb_h.at[idx_n.at[pl.ds(0, QROWS)]],
            nb[u].at[pl.ds(q * QROWS, QROWS)], snm[u]).wait()

    iota = lax.iota(jnp.int32, 16)
    iotap = iota * ROWP

    def colsum(scr):
      cols = [plsc.load_gather(scr, [iotap + cc]) for cc in range(16)]
      while len(cols) > 1:
        cols = [cols[j] + cols[j + 1] for j in range(0, len(cols), 2)]
      return cols[0]

    def row_chunks(ref, row):
      return [ref[row, pl.ds(c * 16, 16)] for c in range(4)]

    def dot4(a, b):
      return (a[0] * b[0] + a[1] * b[1]) + (a[2] * b[2] + a[3] * b[3])

    def compute(g, u):
      t_ref, c_ref, n_ref = tb[u], cb[u], nb[u]
      for i in range(G):
        acc = dot4(row_chunks(t_ref, i), row_chunks(c_ref, i))
        plsc.store_scatter(scra, [iota + i * ROWP], acc)
      plsc.store_scatter(pos_st, [g * G + iota], colsum(scra))

      @pl.loop(0, G // 4)
      def _sub(sub):
        b0 = sub * 4
        tch = [row_chunks(t_ref, b0 + i) for i in range(4)]
        for kc in range(K // 4):
          scr = scrb if kc % 2 else scra
          for cc in range(4):
            for i in range(4):
              nrow = (b0 + i) * K + kc * 4 + cc
              acc = dot4(tch[i], row_chunks(n_ref, nrow))
              plsc.store_scatter(scr, [iota + (cc * 4 + i) * ROWP], acc)
          dst = ((kc * 4 + lax.shift_right_logical(iota, 2)) * S
                 + g * G + b0 + (iota & 3))
          plsc.store_scatter(neg_st, [dst], colsum(scr))

    issue(0, 0)
    issue(1, 1)

    @pl.loop(0, NG // 2)
    def _t(t):
      g = t * 2
      wait(0)
      compute(g, 0)

      @pl.when(g + 2 < NG)
      def _():
        issue(g + 2, 0)

      wait(1)
      compute(g + 1, 1)

      @pl.when(g + 3 < NG)
      def _():
        issue(g + 3, 1)

    pltpu.sync_copy(pos_st, pos_o.at[wid])
    pltpu.sync_copy(neg_st, neg_o.at[wid])

  return k(emb, cemb, tgt, ctx, negf)


def _loss_tc(pos2d, neg2d):
  def body(p_ref, n_ref, o_ref):
    p = p_ref[...]
    n = n_ref[...]
    lp = jnp.minimum(p, 0.0) - jnp.log1p(jnp.exp(-jnp.abs(p)))
    ln = jnp.minimum(-n, 0.0) - jnp.log1p(jnp.exp(-jnp.abs(n)))
    o_ref[0, 0] = -(jnp.sum(lp) + jnp.sum(ln)) / B

  return pl.pallas_call(
      body,
      out_shape=jax.ShapeDtypeStruct((1, 1), jnp.float32),
      out_specs=pl.BlockSpec(memory_space=pltpu.SMEM),
  )(pos2d, neg2d)


def kernel(embeddings, context_embeddings, target, context, negative_samples):
  tgt = target.astype(jnp.int32)
  ctx = context.astype(jnp.int32)
  negf = negative_samples.astype(jnp.int32).reshape(-1)
  pos_l, neg_l = _sc_logits(embeddings, context_embeddings, tgt, ctx, negf)
  loss = _loss_tc(pos_l.reshape(B // 128, 128), neg_l.reshape(B * K // 128, 128))
  return loss.reshape(())
